# Initial kernel scaffold; baseline (speedup 1.0000x reference)
import jax
import jax.numpy as jnp
from jax.experimental import pallas as pl


def kernel(agent_h, team_idx, n_teams, W1, b1, W2, b2, Wo, bo):
    raise NotImplementedError("write your pallas kernel here")


# speedup vs baseline: 1.3831x; 1.3831x over previous
import jax
import jax.numpy as jnp
from jax.experimental import pallas as pl

_T = 10000
_BN = 1280
_C = 1280
_TP = 10240
_NC = _TP // _C


def _logits_segmax_k(ti_ref, ah_ref, w1_ref, b1_ref, w2_ref, b2_ref,
                     lg_ref, mx_ref):
    @pl.when(pl.program_id(0) == 0)
    def _init():
        mx_ref[...] = jnp.full(mx_ref.shape, -jnp.inf, jnp.float32)

    h = jnp.tanh(jnp.dot(ah_ref[...], w1_ref[...],
                         preferred_element_type=jnp.float32) + b1_ref[...])
    lg = jnp.dot(h, w2_ref[...], preferred_element_type=jnp.float32) + b2_ref[...]
    lg_ref[...] = lg

    ti = ti_ref[...]
    tmin = jnp.min(ti)
    tmax = jnp.max(ti)
    for c in range(_NC):
        lo = c * _C

        @pl.when(jnp.logical_and(tmax >= lo, tmin < lo + _C))
        def _upd(lo=lo):
            ids = lo + jax.lax.broadcasted_iota(jnp.int32, (_BN, _C), 1)
            m = ti == ids
            vals = jnp.where(m, lg, -jnp.inf)
            cmax = jnp.max(vals, axis=0, keepdims=True)
            mx_ref[:, lo:lo + _C] = jnp.maximum(mx_ref[:, lo:lo + _C], cmax)


def _exp_segsum_k(ti_ref, lg_ref, mx_ref, e_ref, ss_ref):
    @pl.when(pl.program_id(0) == 0)
    def _init():
        ss_ref[...] = jnp.zeros(ss_ref.shape, jnp.float32)

    ti = ti_ref[...]
    lg = lg_ref[...]
    g = jnp.zeros(lg.shape, jnp.float32)
    for c in range(_NC):
        lo = c * _C
        ids = lo + jax.lax.broadcasted_iota(jnp.int32, (_BN, _C), 1)
        m = ti == ids
        g = g + jnp.sum(jnp.where(m, mx_ref[:, lo:lo + _C], 0.0),
                        axis=1, keepdims=True)
    e = jnp.exp(lg - g)
    e_ref[...] = e

    tmin = jnp.min(ti)
    tmax = jnp.max(ti)
    for c in range(_NC):
        lo = c * _C

        @pl.when(jnp.logical_and(tmax >= lo, tmin < lo + _C))
        def _upd(lo=lo):
            ids = lo + jax.lax.broadcasted_iota(jnp.int32, (_BN, _C), 1)
            m = ti == ids
            csum = jnp.sum(jnp.where(m, e, 0.0), axis=0, keepdims=True)
            ss_ref[:, lo:lo + _C] = ss_ref[:, lo:lo + _C] + csum


def _pool_k(ti_ref, tr_ref, ah_ref, e_ref, ss_ref, w_ref, th_ref):
    @pl.when(pl.program_id(0) == 0)
    def _init():
        th_ref[...] = jnp.zeros(th_ref.shape, jnp.float32)

    ti = ti_ref[...]
    e = e_ref[...]
    s = jnp.zeros(e.shape, jnp.float32)
    for c in range(_NC):
        lo = c * _C
        ids = lo + jax.lax.broadcasted_iota(jnp.int32, (_BN, _C), 1)
        m = ti == ids
        s = s + jnp.sum(jnp.where(m, ss_ref[:, lo:lo + _C], 0.0),
                        axis=1, keepdims=True)
    w = e / (s + 1e-8)
    w_ref[...] = w

    wh = ah_ref[...] * w
    tr = tr_ref[...]
    tmin = jnp.min(ti)
    tmax = jnp.max(ti)
    for c in range(_NC):
        lo = c * _C

        @pl.when(jnp.logical_and(tmax >= lo, tmin < lo + _C))
        def _upd(lo=lo):
            ids = lo + jax.lax.broadcasted_iota(jnp.int32, (_C, _BN), 0)
            oh = (ids == tr).astype(jnp.float32)
            contrib = jnp.dot(oh, wh, preferred_element_type=jnp.float32)
            th_ref[lo:lo + _C, :] = th_ref[lo:lo + _C, :] + contrib


def _proj_k(th_ref, wo_ref, bo_ref, out_ref):
    out_ref[...] = jax.nn.relu(
        jnp.dot(th_ref[...], wo_ref[...], preferred_element_type=jnp.float32)
        + bo_ref[...])


def kernel(agent_h, team_idx, n_teams, W1, b1, W2, b2, Wo, bo):
    N, D = agent_h.shape
    H = W1.shape[1]
    nb = N // _BN
    ti = jnp.minimum(team_idx, n_teams - 1).astype(jnp.int32)
    ti_col = ti.reshape(N, 1)
    ti_row = ti.reshape(1, N)
    b1r = b1.reshape(1, H)
    b2r = b2.reshape(1, 1)
    bor = bo.reshape(1, D)

    lg, mx = pl.pallas_call(
        _logits_segmax_k,
        grid=(nb,),
        in_specs=[
            pl.BlockSpec((_BN, 1), lambda i: (i, 0)),
            pl.BlockSpec((_BN, D), lambda i: (i, 0)),
            pl.BlockSpec((D, H), lambda i: (0, 0)),
            pl.BlockSpec((1, H), lambda i: (0, 0)),
            pl.BlockSpec((H, 1), lambda i: (0, 0)),
            pl.BlockSpec((1, 1), lambda i: (0, 0)),
        ],
        out_specs=[
            pl.BlockSpec((_BN, 1), lambda i: (i, 0)),
            pl.BlockSpec((1, _TP), lambda i: (0, 0)),
        ],
        out_shape=[
            jax.ShapeDtypeStruct((N, 1), jnp.float32),
            jax.ShapeDtypeStruct((1, _TP), jnp.float32),
        ],
    )(ti_col, agent_h, W1, b1r, W2, b2r)

    e, ss = pl.pallas_call(
        _exp_segsum_k,
        grid=(nb,),
        in_specs=[
            pl.BlockSpec((_BN, 1), lambda i: (i, 0)),
            pl.BlockSpec((_BN, 1), lambda i: (i, 0)),
            pl.BlockSpec((1, _TP), lambda i: (0, 0)),
        ],
        out_specs=[
            pl.BlockSpec((_BN, 1), lambda i: (i, 0)),
            pl.BlockSpec((1, _TP), lambda i: (0, 0)),
        ],
        out_shape=[
            jax.ShapeDtypeStruct((N, 1), jnp.float32),
            jax.ShapeDtypeStruct((1, _TP), jnp.float32),
        ],
    )(ti_col, lg, mx)

    w, th = pl.pallas_call(
        _pool_k,
        grid=(nb,),
        in_specs=[
            pl.BlockSpec((_BN, 1), lambda i: (i, 0)),
            pl.BlockSpec((1, _BN), lambda i: (0, i)),
            pl.BlockSpec((_BN, D), lambda i: (i, 0)),
            pl.BlockSpec((_BN, 1), lambda i: (i, 0)),
            pl.BlockSpec((1, _TP), lambda i: (0, 0)),
        ],
        out_specs=[
            pl.BlockSpec((_BN, 1), lambda i: (i, 0)),
            pl.BlockSpec((_TP, D), lambda i: (0, 0)),
        ],
        out_shape=[
            jax.ShapeDtypeStruct((N, 1), jnp.float32),
            jax.ShapeDtypeStruct((_TP, D), jnp.float32),
        ],
    )(ti_col, ti_row, agent_h, e, ss)

    team_h = pl.pallas_call(
        _proj_k,
        grid=(_TP // _C,),
        in_specs=[
            pl.BlockSpec((_C, D), lambda i: (i, 0)),
            pl.BlockSpec((D, D), lambda i: (0, 0)),
            pl.BlockSpec((1, D), lambda i: (0, 0)),
        ],
        out_specs=pl.BlockSpec((_C, D), lambda i: (i, 0)),
        out_shape=jax.ShapeDtypeStruct((_TP, D), jnp.float32),
    )(th, Wo, bor)

    return (team_h[:_T], w.reshape(N))


# speedup vs baseline: 3.2741x; 2.3673x over previous
import functools

import jax
import jax.numpy as jnp
from jax.experimental import pallas as pl
from jax.experimental.pallas import tpu as pltpu

_T = 10000
_BN = 1280
_C = 640
_TP = 10240
_NC = _TP // _C


def _logits_segmax_k(ti_ref, ah_ref, w1_ref, b1_ref, w2_ref, b2_ref,
                     lg_ref, mx_ref):
    @pl.when(pl.program_id(0) == 0)
    def _init():
        mx_ref[...] = jnp.full(mx_ref.shape, -jnp.inf, jnp.float32)

    h = jnp.tanh(jnp.dot(ah_ref[...], w1_ref[...],
                         preferred_element_type=jnp.float32) + b1_ref[...])
    lg = jnp.dot(h, w2_ref[...], preferred_element_type=jnp.float32) + b2_ref[...]
    lg_ref[...] = lg

    ti = ti_ref[...]
    tmin = jnp.min(ti)
    tmax = jnp.max(ti)
    for c in range(_NC):
        lo = c * _C

        @pl.when(jnp.logical_and(tmax >= lo, tmin < lo + _C))
        def _upd(lo=lo):
            ids = lo + jax.lax.broadcasted_iota(jnp.int32, (_BN, _C), 1)
            m = ti == ids
            vals = jnp.where(m, lg, -jnp.inf)
            cmax = jnp.max(vals, axis=0, keepdims=True)
            mx_ref[:, lo:lo + _C] = jnp.maximum(mx_ref[:, lo:lo + _C], cmax)


def _exp_segsum_k(ti_ref, lg_ref, mx_ref, e_ref, ss_ref, g_ref):
    @pl.when(pl.program_id(0) == 0)
    def _init():
        ss_ref[...] = jnp.zeros(ss_ref.shape, jnp.float32)

    ti = ti_ref[...]
    lg = lg_ref[...]
    tmin = jnp.min(ti)
    tmax = jnp.max(ti)

    g_ref[...] = jnp.zeros(g_ref.shape, jnp.float32)
    for c in range(_NC):
        lo = c * _C

        @pl.when(jnp.logical_and(tmax >= lo, tmin < lo + _C))
        def _gather(lo=lo):
            ids = lo + jax.lax.broadcasted_iota(jnp.int32, (_BN, _C), 1)
            m = ti == ids
            g_ref[...] = g_ref[...] + jnp.sum(
                jnp.where(m, mx_ref[:, lo:lo + _C], 0.0), axis=1, keepdims=True)

    e = jnp.exp(lg - g_ref[...])
    e_ref[...] = e

    for c in range(_NC):
        lo = c * _C

        @pl.when(jnp.logical_and(tmax >= lo, tmin < lo + _C))
        def _scatter(lo=lo):
            ids = lo + jax.lax.broadcasted_iota(jnp.int32, (_BN, _C), 1)
            m = ti == ids
            csum = jnp.sum(jnp.where(m, e, 0.0), axis=0, keepdims=True)
            ss_ref[:, lo:lo + _C] = ss_ref[:, lo:lo + _C] + csum


def _pool_k(ti_ref, tr_ref, ah_ref, e_ref, ss_ref, wo_ref, bo_ref,
            w_ref, out_ref, s_ref, th_ref, *, nb):
    @pl.when(pl.program_id(0) == 0)
    def _init():
        th_ref[...] = jnp.zeros(th_ref.shape, jnp.float32)

    ti = ti_ref[...]
    e = e_ref[...]
    tmin = jnp.min(ti)
    tmax = jnp.max(ti)

    s_ref[...] = jnp.zeros(s_ref.shape, jnp.float32)
    for c in range(_NC):
        lo = c * _C

        @pl.when(jnp.logical_and(tmax >= lo, tmin < lo + _C))
        def _gather(lo=lo):
            ids = lo + jax.lax.broadcasted_iota(jnp.int32, (_BN, _C), 1)
            m = ti == ids
            s_ref[...] = s_ref[...] + jnp.sum(
                jnp.where(m, ss_ref[:, lo:lo + _C], 0.0), axis=1, keepdims=True)

    w = e / (s_ref[...] + 1e-8)
    w_ref[...] = w

    wh = ah_ref[...] * w
    tr = tr_ref[...]
    for c in range(_NC):
        lo = c * _C

        @pl.when(jnp.logical_and(tmax >= lo, tmin < lo + _C))
        def _scatter(lo=lo):
            ids = lo + jax.lax.broadcasted_iota(jnp.int32, (_C, _BN), 0)
            oh = (ids == tr).astype(jnp.float32)
            contrib = jnp.dot(oh, wh, preferred_element_type=jnp.float32)
            th_ref[lo:lo + _C, :] = th_ref[lo:lo + _C, :] + contrib

    @pl.when(pl.program_id(0) == nb - 1)
    def _proj():
        out_ref[...] = jax.nn.relu(
            jnp.dot(th_ref[...], wo_ref[...], preferred_element_type=jnp.float32)
            + bo_ref[...])


def kernel(agent_h, team_idx, n_teams, W1, b1, W2, b2, Wo, bo):
    N, D = agent_h.shape
    H = W1.shape[1]
    nb = N // _BN
    ti = jnp.minimum(team_idx, n_teams - 1).astype(jnp.int32)
    ti_col = ti.reshape(N, 1)
    ti_row = ti.reshape(1, N)
    b1r = b1.reshape(1, H)
    b2r = b2.reshape(1, 1)
    bor = bo.reshape(1, D)

    lg, mx = pl.pallas_call(
        _logits_segmax_k,
        grid=(nb,),
        in_specs=[
            pl.BlockSpec((_BN, 1), lambda i: (i, 0)),
            pl.BlockSpec((_BN, D), lambda i: (i, 0)),
            pl.BlockSpec((D, H), lambda i: (0, 0)),
            pl.BlockSpec((1, H), lambda i: (0, 0)),
            pl.BlockSpec((H, 1), lambda i: (0, 0)),
            pl.BlockSpec((1, 1), lambda i: (0, 0)),
        ],
        out_specs=[
            pl.BlockSpec((_BN, 1), lambda i: (i, 0)),
            pl.BlockSpec((1, _TP), lambda i: (0, 0)),
        ],
        out_shape=[
            jax.ShapeDtypeStruct((N, 1), jnp.float32),
            jax.ShapeDtypeStruct((1, _TP), jnp.float32),
        ],
    )(ti_col, agent_h, W1, b1r, W2, b2r)

    e, ss = pl.pallas_call(
        _exp_segsum_k,
        grid=(nb,),
        in_specs=[
            pl.BlockSpec((_BN, 1), lambda i: (i, 0)),
            pl.BlockSpec((_BN, 1), lambda i: (i, 0)),
            pl.BlockSpec((1, _TP), lambda i: (0, 0)),
        ],
        out_specs=[
            pl.BlockSpec((_BN, 1), lambda i: (i, 0)),
            pl.BlockSpec((1, _TP), lambda i: (0, 0)),
        ],
        out_shape=[
            jax.ShapeDtypeStruct((N, 1), jnp.float32),
            jax.ShapeDtypeStruct((1, _TP), jnp.float32),
        ],
        scratch_shapes=[pltpu.VMEM((_BN, 1), jnp.float32)],
    )(ti_col, lg, mx)

    w, team_h = pl.pallas_call(
        functools.partial(_pool_k, nb=nb),
        grid=(nb,),
        in_specs=[
            pl.BlockSpec((_BN, 1), lambda i: (i, 0)),
            pl.BlockSpec((1, _BN), lambda i: (0, i)),
            pl.BlockSpec((_BN, D), lambda i: (i, 0)),
            pl.BlockSpec((_BN, 1), lambda i: (i, 0)),
            pl.BlockSpec((1, _TP), lambda i: (0, 0)),
            pl.BlockSpec((D, D), lambda i: (0, 0)),
            pl.BlockSpec((1, D), lambda i: (0, 0)),
        ],
        out_specs=[
            pl.BlockSpec((_BN, 1), lambda i: (i, 0)),
            pl.BlockSpec((_TP, D), lambda i: (0, 0)),
        ],
        out_shape=[
            jax.ShapeDtypeStruct((N, 1), jnp.float32),
            jax.ShapeDtypeStruct((_TP, D), jnp.float32),
        ],
        scratch_shapes=[
            pltpu.VMEM((_BN, 1), jnp.float32),
            pltpu.VMEM((_TP, D), jnp.float32),
        ],
    )(ti_col, ti_row, agent_h, e, ss, Wo, bor)

    return (team_h[:_T], w.reshape(N))
